# baseline (speedup 1.0000x reference)
import functools

import jax
import jax.numpy as jnp
from jax import lax
from jax.experimental import pallas as pl
from jax.experimental.pallas import tpu as pltpu
from jax.experimental.pallas import tpu_sc as plsc

B = 32
V = 25000
F = 50000
L = 16
NC, NS = 2, 16

CH = 2048
NFULL = F // CH
REM = F - NFULL * CH


def _body(vT, fT, oT, verts_v, faces_v, out_v, verts_sp, faces_sp, out_sp):
    c = lax.axis_index("c")
    s = lax.axis_index("s")
    b0 = c * NS

    for h in range(NS):
        @pl.when(s == 0)
        def _():
            for k in range(3):
                pltpu.sync_copy(vT.at[k, b0 + h], verts_sp.at[k])

        plsc.subcore_barrier()

        @pl.when(s == h)
        def _():
            for k in range(3):
                pltpu.sync_copy(verts_sp.at[k], verts_v.at[k])

        plsc.subcore_barrier()

    def do_chunk(f0, w):
        nj = w // L

        for j in range(3):
            @pl.when(s == j)
            def _():
                pltpu.sync_copy(
                    fT.at[j, pl.ds(b0, NS), pl.ds(f0, w)],
                    faces_sp.at[j, :, pl.ds(0, w)],
                )

        plsc.subcore_barrier()
        pltpu.sync_copy(
            faces_sp.at[:, s, pl.ds(0, w)], faces_v.at[:, pl.ds(0, w)]
        )

        @plsc.parallel_loop(0, nj, step=1, unroll=4)
        def _loop(i):
            for j in range(3):
                fj = faces_v[j, pl.ds(i * L, L)]
                for k in range(3):
                    row = jnp.full((L,), k, jnp.int32)
                    vals = plsc.load_gather(verts_v, [row, fj])
                    out_v[3 * j + k, pl.ds(i * L, L)] = vals

        pltpu.sync_copy(
            out_v.at[:, pl.ds(0, w)], out_sp.at[:, s, pl.ds(0, w)]
        )
        plsc.subcore_barrier()

        for r in range(9):
            @pl.when(s == r)
            def _():
                pltpu.sync_copy(
                    out_sp.at[r, :, pl.ds(0, w)],
                    oT.at[r, pl.ds(b0, NS), pl.ds(f0, w)],
                )

    def chunk_step(ci, carry):
        do_chunk(pl.multiple_of(ci * CH, 128), CH)
        return carry

    lax.fori_loop(0, NFULL, chunk_step, 0)
    if REM:
        do_chunk(NFULL * CH, REM)


@functools.partial(jax.jit, static_argnames=())
def kernel(vertices, faces):
    vT = vertices.transpose(2, 0, 1)
    fT = faces.transpose(2, 0, 1)
    mesh = plsc.VectorSubcoreMesh(
        core_axis_name="c", subcore_axis_name="s", num_cores=NC, num_subcores=NS
    )
    outT = pl.kernel(
        _body,
        out_type=jax.ShapeDtypeStruct((9, B, F), jnp.float32),
        mesh=mesh,
        compiler_params=pltpu.CompilerParams(
            needs_layout_passes=False, use_tc_tiling_on_sc=False
        ),
        scratch_types=[
            pltpu.VMEM((3, V), jnp.float32),
            pltpu.VMEM((3, CH), jnp.int32),
            pltpu.VMEM((9, CH), jnp.float32),
            pltpu.VMEM_SHARED((3, V), jnp.float32),
            pltpu.VMEM_SHARED((3, NS, CH), jnp.int32),
            pltpu.VMEM_SHARED((9, NS, CH), jnp.float32),
        ],
    )(vT, fT)
    return outT.transpose(1, 2, 0)


# speedup vs baseline: 1.1083x; 1.1083x over previous
import functools

import jax
import jax.numpy as jnp
from jax import lax
from jax.experimental import pallas as pl
from jax.experimental.pallas import tpu as pltpu
from jax.experimental.pallas import tpu_sc as plsc

B = 32
V = 25000
F = 50000
L = 16
NC, NS = 2, 16

CH = 1536
NFULL = F // CH
REM = F - NFULL * CH


def _body(vT, fT, oT, verts_v, faces_v, out_v, verts_sp, faces_sp, out_sp):
    c = lax.axis_index("c")
    s = lax.axis_index("s")
    b0 = c * NS

    for h in range(8):
        @pl.when(s == 0)
        def _():
            for k in range(3):
                pltpu.sync_copy(
                    vT.at[k, pl.ds(b0 + 2 * h, 2), :], verts_sp.at[k]
                )

        plsc.subcore_barrier()

        @pl.when((s >= 2 * h) & (s < 2 * h + 2))
        def _():
            for k in range(3):
                pltpu.sync_copy(verts_sp.at[k, s - 2 * h], verts_v.at[k])

        plsc.subcore_barrier()

    def do_chunk(f0, w):
        nj = w // L

        for j in range(3):
            @pl.when(s == j)
            def _():
                pltpu.sync_copy(
                    fT.at[j, pl.ds(b0, NS), pl.ds(f0, w)],
                    faces_sp.at[j, :, pl.ds(0, w)],
                )

        plsc.subcore_barrier()
        pltpu.sync_copy(
            faces_sp.at[:, s, pl.ds(0, w)], faces_v.at[:, pl.ds(0, w)]
        )

        @plsc.parallel_loop(0, nj, step=1, unroll=8)
        def _loop(i):
            for j in range(3):
                fj = faces_v[j, pl.ds(i * L, L)]
                for k in range(3):
                    row = jnp.full((L,), k, jnp.int32)
                    vals = plsc.load_gather(verts_v, [row, fj])
                    out_v[3 * j + k, pl.ds(i * L, L)] = vals

        pltpu.sync_copy(
            out_v.at[:, pl.ds(0, w)], out_sp.at[:, s, pl.ds(0, w)]
        )
        plsc.subcore_barrier()

        for r in range(9):
            @pl.when(s == r)
            def _():
                pltpu.sync_copy(
                    out_sp.at[r, :, pl.ds(0, w)],
                    oT.at[r, pl.ds(b0, NS), pl.ds(f0, w)],
                )

    def chunk_step(ci, carry):
        do_chunk(pl.multiple_of(ci * CH, 128), CH)
        return carry

    lax.fori_loop(0, NFULL, chunk_step, 0)
    if REM:
        do_chunk(NFULL * CH, REM)


@functools.partial(jax.jit, static_argnames=())
def kernel(vertices, faces):
    vT = vertices.transpose(2, 0, 1)
    fT = faces.transpose(2, 0, 1)
    mesh = plsc.VectorSubcoreMesh(
        core_axis_name="c", subcore_axis_name="s", num_cores=NC, num_subcores=NS
    )
    outT = pl.kernel(
        _body,
        out_type=jax.ShapeDtypeStruct((9, B, F), jnp.float32),
        mesh=mesh,
        compiler_params=pltpu.CompilerParams(
            needs_layout_passes=False, use_tc_tiling_on_sc=False
        ),
        scratch_types=[
            pltpu.VMEM((3, V), jnp.float32),
            pltpu.VMEM((3, CH), jnp.int32),
            pltpu.VMEM((9, CH), jnp.float32),
            pltpu.VMEM_SHARED((3, 2, V), jnp.float32),
            pltpu.VMEM_SHARED((3, NS, CH), jnp.int32),
            pltpu.VMEM_SHARED((9, NS, CH), jnp.float32),
        ],
    )(vT, fT)
    return outT.transpose(1, 2, 0)


# speedup vs baseline: 1.1988x; 1.0817x over previous
import functools

import jax
import jax.numpy as jnp
from jax import lax
from jax.experimental import pallas as pl
from jax.experimental.pallas import tpu as pltpu
from jax.experimental.pallas import tpu_sc as plsc

B = 32
V = 25000
F = 50000
L = 16
NC, NS = 2, 16

CH = 1536
NFULL = F // CH
REM = F - NFULL * CH


def _body(vT, fT, oT, verts_v, faces_v, out_v, verts_sp, faces_sp, out_sp):
    c = lax.axis_index("c")
    s = lax.axis_index("s")
    b0 = c * NS

    for h in range(4):
        for k in range(3):
            @pl.when(s == k)
            def _():
                pltpu.sync_copy(
                    vT.at[k, pl.ds(b0 + 4 * h, 4), :], verts_sp.at[k]
                )

        plsc.subcore_barrier()

        @pl.when((s >= 4 * h) & (s < 4 * h + 4))
        def _():
            for k in range(3):
                pltpu.sync_copy(verts_sp.at[k, s - 4 * h], verts_v.at[k])

        plsc.subcore_barrier()

    def do_chunk(f0, w):
        nj = w // L

        for j in range(3):
            @pl.when(s == j)
            def _():
                pltpu.sync_copy(
                    fT.at[j, pl.ds(b0, NS), pl.ds(f0, w)],
                    faces_sp.at[:, j, pl.ds(0, w)],
                )

        plsc.subcore_barrier()
        pltpu.sync_copy(
            faces_sp.at[s, :, pl.ds(0, w)], faces_v.at[:, pl.ds(0, w)]
        )

        @plsc.parallel_loop(0, nj, step=1, unroll=8)
        def _loop(i):
            for j in range(3):
                fj = faces_v[j, pl.ds(i * L, L)]
                for k in range(3):
                    row = jnp.full((L,), k, jnp.int32)
                    vals = plsc.load_gather(verts_v, [row, fj])
                    out_v[3 * j + k, pl.ds(i * L, L)] = vals

        pltpu.sync_copy(
            out_v.at[:, pl.ds(0, w)], out_sp.at[s, :, pl.ds(0, w)]
        )
        plsc.subcore_barrier()

        for r in range(9):
            @pl.when(s == r)
            def _():
                pltpu.sync_copy(
                    out_sp.at[:, r, pl.ds(0, w)],
                    oT.at[r, pl.ds(b0, NS), pl.ds(f0, w)],
                )

    def chunk_step(ci, carry):
        do_chunk(pl.multiple_of(ci * CH, 128), CH)
        return carry

    lax.fori_loop(0, NFULL, chunk_step, 0)
    if REM:
        do_chunk(NFULL * CH, REM)


@functools.partial(jax.jit, static_argnames=())
def kernel(vertices, faces):
    vT = vertices.transpose(2, 0, 1)
    fT = faces.transpose(2, 0, 1)
    mesh = plsc.VectorSubcoreMesh(
        core_axis_name="c", subcore_axis_name="s", num_cores=NC, num_subcores=NS
    )
    outT = pl.kernel(
        _body,
        out_type=jax.ShapeDtypeStruct((9, B, F), jnp.float32),
        mesh=mesh,
        compiler_params=pltpu.CompilerParams(
            needs_layout_passes=False, use_tc_tiling_on_sc=False
        ),
        scratch_types=[
            pltpu.VMEM((3, V), jnp.float32),
            pltpu.VMEM((3, CH), jnp.int32),
            pltpu.VMEM((9, CH), jnp.float32),
            pltpu.VMEM_SHARED((3, 4, V), jnp.float32),
            pltpu.VMEM_SHARED((NS, 3, CH), jnp.int32),
            pltpu.VMEM_SHARED((NS, 9, CH), jnp.float32),
        ],
    )(vT, fT)
    return outT.transpose(1, 2, 0)
